# baseline (speedup 1.0000x reference)
import jax
import jax.numpy as jnp
from jax import lax
from jax.experimental import pallas as pl
from jax.experimental.pallas import tpu as pltpu
from jax.experimental.pallas import tpu_sc as plsc

H = 512
W = 512
C = 96
SH = 36
SW = 36
L = 16
NCV = C // L
ROW = W * C
GROW = SW * C

_mesh = plsc.VectorSubcoreMesh(core_axis_name="c", subcore_axis_name="s",
                               num_cores=2, num_subcores=16)


def _zero_buf(ref, nwords):
    z = jnp.zeros((L,), jnp.float32)

    def body(i, _):
        ref[pl.ds(i * L, L)] = z
        return 0

    lax.fori_loop(0, nwords // L, body, 0)


def _pool_row_into_slab(buf, slab):
    def edge(x0, nx, col):
        for cv in range(NCV):
            a = buf[0, x0, pl.ds(cv * L, L)]
            for j in range(1, nx):
                a = a + buf[0, x0 + j, pl.ds(cv * L, L)]
            plsc.addupdate(slab.at[pl.ds(col * C + cv * L, L)], a)

    edge(0, 8, 2)
    edge(504, 8, 34)

    def body(xb, _):
        x0 = xb * 16 - 8
        col = (xb + 2) * C
        for cv in range(NCV):
            a = buf[0, x0, pl.ds(cv * L, L)]
            for j in range(1, 16):
                a = a + buf[0, x0 + j, pl.ds(cv * L, L)]
            plsc.addupdate(slab.at[pl.ds(col + cv * L, L)], a)
        return 0

    lax.fori_loop(1, 32, body, 0)


def _conv5_row(slab, fs):
    w5 = (0.0625, 0.25, 0.375, 0.25, 0.0625)
    z = jnp.zeros((L,), jnp.float32)
    for k in (0, 1, 35):
        for cv in range(NCV):
            fs[0, k, pl.ds(cv * L, L)] = z

    def body(k, _):
        base = (k - 2) * C
        for cv in range(NCV):
            a = slab[pl.ds(base + cv * L, L)] * w5[0]
            for d in range(1, 5):
                a = a + slab[pl.ds(base + d * C + cv * L, L)] * w5[d]
            fs[0, k, pl.ds(cv * L, L)] = a
        return 0

    lax.fori_loop(2, 34, body, 0)
    for cv in range(NCV):
        a = (slab[pl.ds(32 * C + cv * L, L)] * 0.0625
             + slab[pl.ds(33 * C + cv * L, L)] * 0.25
             + slab[pl.ds(34 * C + cv * L, L)] * 0.3125)
        fs[0, 34, pl.ds(cv * L, L)] = a


def _fused_body(inp_hbm, out_hbm, buf, slab, fs, fbuf, rbuf, fsh):
    cid = lax.axis_index("c")
    sid = lax.axis_index("s")

    slot_base = 17 * cid

    def do_bin(y0, nrows, slot):
        _zero_buf(slab, GROW)

        def row_body(ry, _):
            pltpu.sync_copy(inp_hbm.at[pl.ds(y0 + ry, 1)], buf)
            _pool_row_into_slab(buf, slab)
            return 0

        lax.fori_loop(0, nrows, row_body, 0)
        _conv5_row(slab, fs)
        pltpu.sync_copy(fs, fsh.at[pl.ds(slot_base + slot, 1)])

    @pl.when(jnp.logical_and(cid == 0, sid == 0))
    def _():
        do_bin(0, 8, 0)
        do_bin(248, 16, 16)

    @pl.when(jnp.logical_and(cid == 1, sid == 0))
    def _():
        do_bin(248, 16, 0)
        do_bin(504, 8, 16)

    @pl.when(sid > 0)
    def _():
        do_bin(16 * (16 * cid + sid) - 8, 16, sid)

    plsc.subcore_barrier()

    pltpu.sync_copy(fsh.at[pl.ds(slot_base + sid, 2)], fbuf)
    p = 16 * cid + sid

    def s_body(s, _):
        ays = s.astype(jnp.float32) * 0.0625

        def r_body(k, _):
            for cv in range(NCV):
                v0 = fbuf[0, k, pl.ds(cv * L, L)]
                v1 = fbuf[1, k, pl.ds(cv * L, L)]
                rbuf[pl.ds(k * C + cv * L, L)] = v0 + (v1 - v0) * ays
            return 0

        lax.fori_loop(2, 35, r_body, 0)

        def q_body(q, _):
            abase = (q + 2) * C
            x0 = q * 16
            for cv in range(NCV):
                a = rbuf[pl.ds(abase + cv * L, L)]
                b = rbuf[pl.ds(abase + C + cv * L, L)]
                d = b - a
                for r in range(16):
                    buf[0, x0 + r, pl.ds(cv * L, L)] = a + d * (r * 0.0625)
            return 0

        lax.fori_loop(0, 32, q_body, 0)
        pltpu.sync_copy(buf, out_hbm.at[pl.ds(16 * p + s, 1)])
        return 0

    lax.fori_loop(0, 16, s_body, 0)


_fused = pl.kernel(
    _fused_body,
    out_type=jax.ShapeDtypeStruct((H, W, C), jnp.float32),
    mesh=_mesh,
    scratch_types=[
        pltpu.VMEM((1, W, C), jnp.float32),
        pltpu.VMEM((GROW,), jnp.float32),
        pltpu.VMEM((1, SW, C), jnp.float32),
        pltpu.VMEM((2, SW, C), jnp.float32),
        pltpu.VMEM((GROW,), jnp.float32),
        pltpu.VMEM_SHARED((34, SW, C), jnp.float32),
    ],
)


def kernel(inp):
    return _fused(inp)


# speedup vs baseline: 1.3546x; 1.3546x over previous
import jax
import jax.numpy as jnp
from jax import lax
from jax.experimental import pallas as pl
from jax.experimental.pallas import tpu as pltpu
from jax.experimental.pallas import tpu_sc as plsc

H = 512
W = 512
HW = W // 2
C = 96
SH = 36
SW = 36
L = 16
NCV = C // L
GROW = SW * C

_mesh = plsc.VectorSubcoreMesh(core_axis_name="c", subcore_axis_name="s",
                               num_cores=2, num_subcores=16)


def _zero_buf(ref, nwords):
    z = jnp.zeros((L,), jnp.float32)

    def body(i, _):
        ref[pl.ds(i * L, L)] = z
        return 0

    lax.fori_loop(0, nwords // L, body, 0)


def _tree_sum(vs):
    while len(vs) > 1:
        nxt = [vs[i] + vs[i + 1] for i in range(0, len(vs) - 1, 2)]
        if len(vs) % 2:
            nxt.append(vs[-1])
        vs = nxt
    return vs[0]


def _pool_half_into_slab(buf, half, slab):
    def acc(x0, nx, col):
        for cv in range(NCV):
            a = _tree_sum([buf[half, x0 + j, pl.ds(cv * L, L)]
                           for j in range(nx)])
            plsc.addupdate(slab.at[pl.ds(col * C + cv * L, L)], a)

    if half == 0:
        acc(0, 8, 2)
        acc(248, 8, 18)

        def body(xb, _):
            x0 = xb * 16 - 8
            col = (xb + 2) * C
            for cv in range(NCV):
                a = _tree_sum([buf[half, x0 + j, pl.ds(cv * L, L)]
                               for j in range(16)])
                plsc.addupdate(slab.at[pl.ds(col + cv * L, L)], a)
            return 0

        lax.fori_loop(1, 16, body, 0)
    else:
        acc(0, 8, 18)
        acc(248, 8, 34)

        def body(xb, _):
            x0 = xb * 16 - 264
            col = (xb + 2) * C
            for cv in range(NCV):
                a = _tree_sum([buf[half, x0 + j, pl.ds(cv * L, L)]
                               for j in range(16)])
                plsc.addupdate(slab.at[pl.ds(col + cv * L, L)], a)
            return 0

        lax.fori_loop(17, 32, body, 0)


def _conv5_row(slab, fs):
    w5 = (0.0625, 0.25, 0.375, 0.25, 0.0625)
    z = jnp.zeros((L,), jnp.float32)
    for k in (0, 1, 35):
        for cv in range(NCV):
            fs[0, k, pl.ds(cv * L, L)] = z

    def body(k, _):
        base = (k - 2) * C
        for cv in range(NCV):
            a = _tree_sum([slab[pl.ds(base + d * C + cv * L, L)] * w5[d]
                           for d in range(5)])
            fs[0, k, pl.ds(cv * L, L)] = a
        return 0

    lax.fori_loop(2, 34, body, 0)
    for cv in range(NCV):
        a = (slab[pl.ds(32 * C + cv * L, L)] * 0.0625
             + slab[pl.ds(33 * C + cv * L, L)] * 0.25
             + slab[pl.ds(34 * C + cv * L, L)] * 0.3125)
        fs[0, 34, pl.ds(cv * L, L)] = a


def _fused_body(inp_hbm, out_hbm, buf, slab, fs, fbuf, fsh,
                sem_in, sem_out):
    cid = lax.axis_index("c")
    sid = lax.axis_index("s")
    slot_base = 17 * cid

    def in_copy(y, h):
        return pltpu.make_async_copy(
            inp_hbm.at[pl.ds(y, 1), pl.ds(h * HW, HW)],
            buf.at[pl.ds(h, 1)], sem_in)

    def do_bin(y0, nrows, slot):
        _zero_buf(slab, GROW)
        in_copy(y0, 0).start()

        def row_body(ry, _):
            y = y0 + ry
            in_copy(y, 0).wait()
            in_copy(y, 1).start()
            _pool_half_into_slab(buf, 0, slab)
            in_copy(y, 1).wait()

            @pl.when(ry + 1 < nrows)
            def _():
                in_copy(y + 1, 0).start()

            _pool_half_into_slab(buf, 1, slab)
            return 0

        lax.fori_loop(0, nrows, row_body, 0)
        _conv5_row(slab, fs)
        pltpu.sync_copy(fs, fsh.at[pl.ds(slot_base + slot, 1)])

    @pl.when(jnp.logical_and(cid == 0, sid == 0))
    def _():
        do_bin(0, 8, 0)
        do_bin(248, 16, 16)

    @pl.when(jnp.logical_and(cid == 1, sid == 0))
    def _():
        do_bin(248, 16, 0)
        do_bin(504, 8, 16)

    @pl.when(sid > 0)
    def _():
        do_bin(16 * (16 * cid + sid) - 8, 16, sid)

    plsc.subcore_barrier()

    pltpu.sync_copy(fsh.at[pl.ds(slot_base + sid, 2)], fbuf)
    p = 16 * cid + sid

    def out_copy(y, h):
        return pltpu.make_async_copy(
            buf.at[pl.ds(h, 1)],
            out_hbm.at[pl.ds(y, 1), pl.ds(h * HW, HW)], sem_out)

    def assemble_half(h, q_lo):
        def q_body(ql, _):
            q = ql + q_lo
            x0 = ql * 16
            for cv in range(NCV):
                a = fs[0, q + 2, pl.ds(cv * L, L)]
                b = fs[0, q + 3, pl.ds(cv * L, L)]
                d = b - a
                for r in range(16):
                    buf[h, x0 + r, pl.ds(cv * L, L)] = a + d * (r * 0.0625)
            return 0

        lax.fori_loop(0, 16, q_body, 0)

    def s_body(s, _):
        y = 16 * p + s
        ays = s.astype(jnp.float32) * 0.0625

        def r_body(k, _):
            for cv in range(NCV):
                v0 = fbuf[0, k, pl.ds(cv * L, L)]
                v1 = fbuf[1, k, pl.ds(cv * L, L)]
                fs[0, k, pl.ds(cv * L, L)] = v0 + (v1 - v0) * ays
            return 0

        lax.fori_loop(2, 35, r_body, 0)

        for h in (0, 1):
            @pl.when(s > 0)
            def _():
                out_copy(y - 1, h).wait()

            assemble_half(h, 16 * h)
            out_copy(y, h).start()
        return 0

    lax.fori_loop(0, 16, s_body, 0)
    for h in (0, 1):
        out_copy(16 * p + 15, h).wait()


_fused = pl.kernel(
    _fused_body,
    out_type=jax.ShapeDtypeStruct((H, W, C), jnp.float32),
    mesh=_mesh,
    scratch_types=[
        pltpu.VMEM((2, HW, C), jnp.float32),
        pltpu.VMEM((GROW,), jnp.float32),
        pltpu.VMEM((1, SW, C), jnp.float32),
        pltpu.VMEM((2, SW, C), jnp.float32),
        pltpu.VMEM_SHARED((34, SW, C), jnp.float32),
        pltpu.SemaphoreType.DMA,
        pltpu.SemaphoreType.DMA,
    ],
)


def kernel(inp):
    return _fused(inp)
